# baseline (speedup 1.0000x reference)
import functools

import jax
import jax.numpy as jnp
from jax import lax
from jax.experimental import pallas as pl
from jax.experimental.pallas import tpu as pltpu
from jax.experimental.pallas import tpu_sc as plsc

NC, NS, L = 2, 16, 16
NW = NC * NS
ROWS, COLS = 16384, 200
TOTAL = ROWS * COLS
WROWS = ROWS // 2
ROWS_W = ROWS // NW
R2_W = ROWS_W // 2
R2BLK = 32
NBLK = R2_W // R2BLK
BLK = 2 * R2BLK * COLS
APAD = 128

_MESH = plsc.VectorSubcoreMesh(
    core_axis_name="c", subcore_axis_name="s", num_cores=NC, num_subcores=NS
)


@functools.partial(
    pl.kernel,
    out_type=jax.ShapeDtypeStruct((2 * WROWS, 128), jnp.int32),
    mesh=_MESH,
    scratch_types=[
        pltpu.VMEM((BLK + APAD,), jnp.int32),
        pltpu.VMEM((BLK + APAD,), jnp.int32),
        pltpu.VMEM((2 * R2BLK, 128), jnp.int32),
        pltpu.VMEM((2 * R2BLK, 128), jnp.int32),
        pltpu.VMEM((256,), jnp.int32),
        pltpu.SemaphoreType.DMA,
        pltpu.SemaphoreType.DMA,
        pltpu.SemaphoreType.DMA,
        pltpu.SemaphoreType.DMA,
    ],
    compiler_params=pltpu.CompilerParams(needs_layout_passes=False),
)
def _lut_gather(a_hbm, lo_hbm, out_hbm,
                a0, a1, o0, o1, lo_v,
                in_sem0, in_sem1, out_sem0, out_sem1):
    wid = lax.axis_index("s") * NC + lax.axis_index("c")
    base = wid * (ROWS_W * COLS)
    r2_base = wid * R2_W
    a_bufs = (a0, a1)
    o_bufs = (o0, o1)
    in_sems = (in_sem0, in_sem1)
    out_sems = (out_sem0, out_sem1)

    pltpu.sync_copy(lo_hbm, lo_v)

    def start_in(blk, slot):
        off = base + blk * BLK
        return pltpu.async_copy(
            a_hbm.at[pl.ds(off, BLK)],
            a_bufs[slot].at[pl.ds(0, BLK)],
            in_sems[slot],
        )

    def start_out(blk, slot):
        r2 = r2_base + blk * R2BLK
        top = pltpu.async_copy(
            o_bufs[slot].at[pl.ds(0, R2BLK)],
            out_hbm.at[pl.ds(r2, R2BLK)],
            out_sems[slot],
        )
        bot = pltpu.async_copy(
            o_bufs[slot].at[pl.ds(R2BLK, R2BLK)],
            out_hbm.at[pl.ds(WROWS + r2, R2BLK)],
            out_sems[slot],
        )
        return (top, bot)

    def compute(slot):
        a_ref = a_bufs[slot]
        o_ref = o_bufs[slot]
        iota1 = lax.iota(jnp.int32, L)

        @plsc.parallel_loop(0, R2BLK, 1, unroll=2)
        def _body(r2):
            rb = r2 * (2 * COLS)
            for band in range(2):
                for j in range(8):
                    c0 = band * 128 + 16 * j
                    v_lo = plsc.load_gather(a_ref, [iota1 + (rb + c0)])
                    v_hi = plsc.load_gather(a_ref, [iota1 + (rb + c0 + COLS)])
                    c_lo = v_lo + 128
                    c_hi = v_hi + 128
                    if c0 + L > COLS:
                        c_lo = c_lo & 255
                        c_hi = c_hi & 255
                    g_lo = plsc.load_gather(lo_v, [c_lo])
                    g_hi = plsc.load_gather(lo_v, [c_hi]) << 16
                    o_ref[band * R2BLK + r2, pl.ds(16 * j, L)] = g_lo | g_hi

    in_h = {0: start_in(0, 0)}
    out_h = {}
    for blk in range(NBLK):
        slot = blk % 2
        if blk + 1 < NBLK:
            in_h[blk + 1] = start_in(blk + 1, slot ^ 1)
        in_h[blk].wait()
        if blk >= 2:
            for h in out_h[blk - 2]:
                h.wait()
        compute(slot)
        out_h[blk] = start_out(blk, slot)
    for blk in (NBLK - 2, NBLK - 1):
        for h in out_h[blk]:
            h.wait()


def _unpack_body(wt_ref, wb_ref, out_ref):
    t0 = pltpu.bitcast(wt_ref[...], jnp.int16)
    t1 = pltpu.bitcast(wb_ref[...], jnp.int16)
    out_ref[:, 0:128] = t0
    out_ref[:, 128:COLS] = t1[:, : COLS - 128]


TCB = 512

_unpack = pl.pallas_call(
    _unpack_body,
    grid=(WROWS // TCB,),
    in_specs=[
        pl.BlockSpec((TCB, 128), lambda i: (i, 0)),
        pl.BlockSpec((TCB, 128), lambda i: (i + WROWS // TCB, 0)),
    ],
    out_specs=pl.BlockSpec((2 * TCB, COLS), lambda i: (i, 0)),
    out_shape=jax.ShapeDtypeStruct((ROWS, COLS), jnp.int16),
)


def kernel(a, b, table):
    idx_b = jnp.asarray(b, jnp.int32) + 128
    column = lax.dynamic_index_in_dim(table, idx_b, axis=1, keepdims=False)
    lo = column.astype(jnp.int32) & 0xFFFF
    a_flat = a.reshape(TOTAL)
    words = _lut_gather(a_flat, lo)
    return _unpack(words, words)


# speedup vs baseline: 1.0728x; 1.0728x over previous
import functools

import jax
import jax.numpy as jnp
from jax import lax
from jax.experimental import pallas as pl
from jax.experimental.pallas import tpu as pltpu
from jax.experimental.pallas import tpu_sc as plsc

NC, NS, L = 2, 16, 16
NW = NC * NS
ROWS, COLS = 16384, 200
TOTAL = ROWS * COLS
WROWS = ROWS // 2
ROWS_W = ROWS // NW
R2_W = ROWS_W // 2
R2BLK = 32
NBLK = R2_W // R2BLK
BLK = 2 * R2BLK * COLS
APAD = 128

_MESH = plsc.VectorSubcoreMesh(
    core_axis_name="c", subcore_axis_name="s", num_cores=NC, num_subcores=NS
)


@functools.partial(
    pl.kernel,
    out_type=jax.ShapeDtypeStruct((2 * WROWS, 128), jnp.int32),
    mesh=_MESH,
    scratch_types=[
        pltpu.VMEM((BLK + APAD,), jnp.int32),
        pltpu.VMEM((BLK + APAD,), jnp.int32),
        pltpu.VMEM((2 * R2BLK, 128), jnp.int32),
        pltpu.VMEM((2 * R2BLK, 128), jnp.int32),
        pltpu.VMEM((256,), jnp.int32),
        pltpu.SemaphoreType.DMA,
        pltpu.SemaphoreType.DMA,
        pltpu.SemaphoreType.DMA,
        pltpu.SemaphoreType.DMA,
    ],
    compiler_params=pltpu.CompilerParams(needs_layout_passes=False),
)
def _lut_gather(a_hbm, lo_hbm, out_hbm,
                a0, a1, o0, o1, lo_v,
                in_sem0, in_sem1, out_sem0, out_sem1):
    wid = lax.axis_index("s") * NC + lax.axis_index("c")
    base = wid * (ROWS_W * COLS)
    r2_base = wid * R2_W
    a_bufs = (a0, a1)
    o_bufs = (o0, o1)
    in_sems = (in_sem0, in_sem1)
    out_sems = (out_sem0, out_sem1)

    pltpu.sync_copy(lo_hbm, lo_v)

    def start_in(blk, slot):
        off = base + blk * BLK
        return pltpu.async_copy(
            a_hbm.at[pl.ds(off, BLK)],
            a_bufs[slot].at[pl.ds(0, BLK)],
            in_sems[slot],
        )

    def start_out(blk, slot):
        r2 = r2_base + blk * R2BLK
        top = pltpu.async_copy(
            o_bufs[slot].at[pl.ds(0, R2BLK)],
            out_hbm.at[pl.ds(r2, R2BLK)],
            out_sems[slot],
        )
        bot = pltpu.async_copy(
            o_bufs[slot].at[pl.ds(R2BLK, R2BLK)],
            out_hbm.at[pl.ds(WROWS + r2, R2BLK)],
            out_sems[slot],
        )
        return (top, bot)

    def compute(slot):
        a_ref = a_bufs[slot]
        o_ref = o_bufs[slot]
        iota1 = lax.iota(jnp.int32, L)

        @plsc.parallel_loop(0, R2BLK, 1, unroll=2)
        def _body(r2):
            rb = r2 * (2 * COLS)
            for band in range(2):
                for j in range(8):
                    c0 = band * 128 + 16 * j
                    v_lo = plsc.load_gather(a_ref, [iota1 + (rb + c0)])
                    v_hi = plsc.load_gather(a_ref, [iota1 + (rb + c0 + COLS)])
                    c_lo = (v_lo + 128) & 255
                    c_hi = (v_hi + 128) & 255
                    g_lo = plsc.load_gather(lo_v, [c_lo])
                    g_hi = plsc.load_gather(lo_v, [c_hi]) << 16
                    o_ref[band * R2BLK + r2, pl.ds(16 * j, L)] = g_lo | g_hi

    in_h = {0: start_in(0, 0)}
    out_h = {}
    for blk in range(NBLK):
        slot = blk % 2
        if blk + 1 < NBLK:
            in_h[blk + 1] = start_in(blk + 1, slot ^ 1)
        in_h[blk].wait()
        if blk >= 2:
            for h in out_h[blk - 2]:
                h.wait()
        compute(slot)
        out_h[blk] = start_out(blk, slot)
    for blk in (NBLK - 2, NBLK - 1):
        for h in out_h[blk]:
            h.wait()


def _unpack_body(wt_ref, wb_ref, out_ref):
    t0 = pltpu.bitcast(wt_ref[...], jnp.int16)
    t1 = pltpu.bitcast(wb_ref[...], jnp.int16)
    out_ref[:, 0:128] = t0
    out_ref[:, 128:COLS] = t1[:, : COLS - 128]


TCB = 1024

_unpack = pl.pallas_call(
    _unpack_body,
    grid=(WROWS // TCB,),
    in_specs=[
        pl.BlockSpec((TCB, 128), lambda i: (i, 0)),
        pl.BlockSpec((TCB, 128), lambda i: (i + WROWS // TCB, 0)),
    ],
    out_specs=pl.BlockSpec((2 * TCB, COLS), lambda i: (i, 0)),
    out_shape=jax.ShapeDtypeStruct((ROWS, COLS), jnp.int16),
)


def kernel(a, b, table):
    idx_b = jnp.asarray(b, jnp.int32) + 128
    column = lax.dynamic_index_in_dim(table, idx_b, axis=1, keepdims=False)
    lo = column.astype(jnp.int32) & 0xFFFF
    a_flat = a.reshape(TOTAL)
    words = _lut_gather(a_flat, lo)
    return _unpack(words, words)
